# Initial kernel scaffold; baseline (speedup 1.0000x reference)
import jax
import jax.numpy as jnp
from jax.experimental import pallas as pl


def kernel(x, W_in, b_in, g_in, be_in, g1, be1, W1a, b1a, W1b, b1b, g2, be2, W2a, b2a, W2b, b2b, g3, be3, W3a, b3a, W3b, b3b, g_h, be_h, Wh1, bh1, Wh2, bh2):
    raise NotImplementedError("write your pallas kernel here")


# speedup vs baseline: 1.8531x; 1.8531x over previous
import functools

import jax
import jax.numpy as jnp
from jax.experimental import pallas as pl

_H = 128
_K = 8
_THR = 1e-06


def _front_kernel(d_ref, c_ref, uL_ref, uR_ref, fc_ref, vd_ref, cnt_ref):
    d = d_ref[...]
    c = c_ref[...]
    x = d.shape[1]
    dn = jnp.concatenate([d[:, 1:], d[:, :1]], axis=1)
    cn = jnp.concatenate([c[:, 1:], c[:, :1]], axis=1)
    lane = jax.lax.broadcasted_iota(jnp.int32, d.shape, 1)
    disc = (jnp.abs(d - dn) > _THR) & (lane < x - 1)
    cnt = jnp.sum(disc.astype(jnp.int32), axis=1, keepdims=True)
    cnt_ref[...] = jnp.broadcast_to(cnt, cnt_ref.shape)
    fcall = (c + cn) * 0.5
    inf = jnp.float32(jnp.inf)
    score = jnp.where(disc, fcall, inf)
    for k in range(_K):
        m = jnp.min(score, axis=1, keepdims=True)
        cand = jnp.where(score == m, lane, jnp.int32(x))
        fidx = jnp.min(cand, axis=1, keepdims=True)
        sel = lane == fidx
        uL_ref[:, pl.ds(k, 1)] = jnp.sum(jnp.where(sel, d, 0.0), axis=1, keepdims=True)
        uR_ref[:, pl.ds(k, 1)] = jnp.sum(jnp.where(sel, dn, 0.0), axis=1, keepdims=True)
        fc_ref[:, pl.ds(k, 1)] = jnp.sum(jnp.where(sel, fcall, 0.0), axis=1, keepdims=True)
        vd_ref[:, pl.ds(k, 1)] = (m < inf).astype(jnp.float32)
        score = jnp.where(sel, inf, score)


def _ln(h, g, b):
    mu = jnp.mean(h, axis=-1, keepdims=True)
    var = jnp.mean((h - mu) ** 2, axis=-1, keepdims=True)
    return (h - mu) / jnp.sqrt(var + 1e-5) * g + b


def _gelu(h):
    return 0.5 * h * (1.0 + jax.lax.erf(h * 0.7071067811865476))


def _mlp_kernel(uLt_ref, uRt_ref, Win_ref, bin_ref, gin_ref, bein_ref,
                g1_ref, be1_ref, W1a_ref, b1a_ref, W1b_ref, b1b_ref,
                g2_ref, be2_ref, W2a_ref, b2a_ref, W2b_ref, b2b_ref,
                g3_ref, be3_ref, W3a_ref, b3a_ref, W3b_ref, b3b_ref,
                gh_ref, beh_ref, Wh1_ref, bh1_ref, Wh2_ref, bh2_ref,
                o_ref):
    uL = uLt_ref[...]
    uR = uRt_ref[...]
    diff = uL - uR
    feats = (uL, uR, diff, jnp.abs(diff), (uL + uR) * 0.5, jnp.sign(diff))
    Win = Win_ref[...]
    h3 = feats[0][:, :, None] * Win[0][None, None, :]
    for f in range(1, 6):
        h3 = h3 + feats[f][:, :, None] * Win[f][None, None, :]
    kk, bb, hh = h3.shape
    h = h3.reshape(kk * bb, hh) + bin_ref[...]
    h = _gelu(_ln(h, gin_ref[...], bein_ref[...]))
    for (g_r, be_r, Wa_r, ba_r, Wb_r, bb_r) in (
            (g1_ref, be1_ref, W1a_ref, b1a_ref, W1b_ref, b1b_ref),
            (g2_ref, be2_ref, W2a_ref, b2a_ref, W2b_ref, b2b_ref),
            (g3_ref, be3_ref, W3a_ref, b3a_ref, W3b_ref, b3b_ref)):
        r = _ln(h, g_r[...], be_r[...])
        r = _gelu(jnp.dot(r, Wa_r[...], preferred_element_type=jnp.float32) + ba_r[...])
        r = jnp.dot(r, Wb_r[...], preferred_element_type=jnp.float32) + bb_r[...]
        h = h + r
    o = _ln(h, gh_ref[...], beh_ref[...])
    o = _gelu(jnp.dot(o, Wh1_ref[...], preferred_element_type=jnp.float32) + bh1_ref[...])
    o = jnp.dot(o, Wh2_ref[...], preferred_element_type=jnp.float32) + bh2_ref[...]
    o_ref[...] = o


def kernel(x, W_in, b_in, g_in, be_in, g1, be1, W1a, b1a, W1b, b1b,
           g2, be2, W2a, b2a, W2b, b2b, g3, be3, W3a, b3a, W3b, b3b,
           g_h, be_h, Wh1, bh1, Wh2, bh2):
    B, C, X = x.shape
    density = x[:, 0, :]
    coords = x[:, C - 1, :]

    RB = 16
    G = B // RB
    f32 = jnp.float32
    outs = pl.pallas_call(
        _front_kernel,
        grid=(G,),
        in_specs=[pl.BlockSpec((RB, X), lambda i: (i, 0)),
                  pl.BlockSpec((RB, X), lambda i: (i, 0))],
        out_specs=[pl.BlockSpec((RB, _K), lambda i: (i, 0))] * 5,
        out_shape=[jax.ShapeDtypeStruct((B, _K), f32)] * 4
        + [jax.ShapeDtypeStruct((B, _K), jnp.int32)],
    )(density, coords)
    uL, uR, fc, valid, cnt = outs
    front_count = cnt[:, 0]

    uLt = uL.T
    uRt = uR.T
    Wh1p = jnp.pad(Wh1, ((0, 0), (0, _H - Wh1.shape[1])))
    bh1p = jnp.pad(bh1, (0, _H - bh1.shape[0]))
    Wh2p = jnp.pad(Wh2, ((0, _H - Wh2.shape[0]), (0, _H - Wh2.shape[1])))
    bh2p = jnp.pad(bh2, (0, _H - bh2.shape[0]))
    row = lambda v: v.reshape(1, -1)
    o_pad = pl.pallas_call(
        _mlp_kernel,
        out_shape=jax.ShapeDtypeStruct((_K * B, _H), f32),
    )(uLt, uRt, W_in, row(b_in), row(g_in), row(be_in),
      row(g1), row(be1), W1a, row(b1a), W1b, row(b1b),
      row(g2), row(be2), W2a, row(b2a), W2b, row(b2b),
      row(g3), row(be3), W3a, row(b3a), W3b, row(b3b),
      row(g_h), row(be_h), Wh1p, row(bh1p), Wh2p, row(bh2p))

    fp_mlp = o_pad[:, :2].reshape(_K, B, 2).transpose(1, 2, 0)
    fp = jnp.concatenate(
        [fp_mlp, uL[:, None, :], uR[:, None, :], fc[:, None, :], valid[:, None, :]],
        axis=1)
    return (fp, front_count)


# speedup vs baseline: 2.0364x; 1.0989x over previous
import functools

import jax
import jax.numpy as jnp
from jax import lax
from jax.experimental import pallas as pl
from jax.experimental.pallas import tpu as pltpu
from jax.experimental.pallas import tpu_sc as plsc

_H = 128
_K = 8
_THR = 1e-06
_X = 8192
_L = 16
_NW = 32
_RPW = 4


def _front_sc(dens_hbm, coords_hbm, uL_hbm, uR_hbm, fc_hbm, vd_hbm, cnt_hbm,
              d0_r, d1_r, d2_r, d3_r, c0_r, c1_r, c2_r, c3_r,
              uL_st, uR_st, fc_st, vd_st, cnt_st, sem):
    wid = lax.axis_index("s") * 2 + lax.axis_index("c")
    base = wid * _RPW
    dbufs = (d0_r, d1_r, d2_r, d3_r)
    cbufs = (c0_r, c1_r, c2_r, c3_r)
    copies = []
    for r in range(_RPW):
        copies.append(pltpu.async_copy(dens_hbm.at[base + r], dbufs[r].at[pl.ds(0, _X)], sem))
        copies.append(pltpu.async_copy(coords_hbm.at[base + r], cbufs[r].at[pl.ds(0, _X)], sem))
    for cp in copies:
        cp.wait()

    inf = jnp.float32(jnp.inf)
    iota = lax.iota(jnp.int32, _L)
    lo8 = iota < _K

    def body(j, carry):
        bests, bidxs, cnts = carry
        nb, ni, nc = [], [], []
        off = j * _L
        for r in range(_RPW):
            d0 = dbufs[r][pl.ds(off, _L)]
            d1 = dbufs[r][pl.ds(off + 1, _L)]
            c0 = cbufs[r][pl.ds(off, _L)]
            c1 = cbufs[r][pl.ds(off + 1, _L)]
            gidx = off + iota
            disc = (jnp.abs(d0 - d1) > _THR) & (gidx < _X - 1)
            score = jnp.where(disc, (c0 + c1) * 0.5, inf)
            scand, sidx = plsc.sort_key_val(score, gidx)
            mk = jnp.where(lo8, bests[r], lax.rev(scand, (0,)))
            mi = jnp.where(lo8, bidxs[r], lax.rev(sidx, (0,)))
            b2, i2 = plsc.sort_key_val(mk, mi)
            nb.append(b2)
            ni.append(i2)
            nc.append(cnts[r] + disc.astype(jnp.int32))
        return (tuple(nb), tuple(ni), tuple(nc))

    init = (tuple(jnp.full((_L,), inf) for _ in range(_RPW)),
            tuple(jnp.zeros((_L,), jnp.int32) for _ in range(_RPW)),
            tuple(jnp.zeros((_L,), jnp.int32) for _ in range(_RPW)))
    bests, bidxs, cnts = lax.fori_loop(0, _X // _L, body, init)

    for r in range(_RPW):
        bv, bi, cv = bests[r], bidxs[r], cnts[r]
        uLg = plsc.load_gather(dbufs[r], [bi])
        uRg = plsc.load_gather(dbufs[r], [bi + 1])
        cLg = plsc.load_gather(cbufs[r], [bi])
        cRg = plsc.load_gather(cbufs[r], [bi + 1])
        uL_st[r, :] = uLg
        uR_st[r, :] = uRg
        fc_st[r, :] = (cLg + cRg) * 0.5
        vd_st[r, :] = jnp.where(bv < inf, jnp.ones((_L,), jnp.float32),
                                jnp.zeros((_L,), jnp.float32))
        cnt_st[r, :] = lax.broadcast(jnp.sum(cv), (_L,))

    pltpu.sync_copy(uL_st, uL_hbm.at[pl.ds(base, _RPW)])
    pltpu.sync_copy(uR_st, uR_hbm.at[pl.ds(base, _RPW)])
    pltpu.sync_copy(fc_st, fc_hbm.at[pl.ds(base, _RPW)])
    pltpu.sync_copy(vd_st, vd_hbm.at[pl.ds(base, _RPW)])
    pltpu.sync_copy(cnt_st, cnt_hbm.at[pl.ds(base, _RPW)])


def _ln(h, g, b):
    mu = jnp.mean(h, axis=-1, keepdims=True)
    var = jnp.mean((h - mu) ** 2, axis=-1, keepdims=True)
    return (h - mu) / jnp.sqrt(var + 1e-5) * g + b


def _gelu(h):
    return 0.5 * h * (1.0 + jax.lax.erf(h * 0.7071067811865476))


def _mlp_kernel(uLt_ref, uRt_ref, Win_ref, bin_ref, gin_ref, bein_ref,
                g1_ref, be1_ref, W1a_ref, b1a_ref, W1b_ref, b1b_ref,
                g2_ref, be2_ref, W2a_ref, b2a_ref, W2b_ref, b2b_ref,
                g3_ref, be3_ref, W3a_ref, b3a_ref, W3b_ref, b3b_ref,
                gh_ref, beh_ref, Wh1_ref, bh1_ref, Wh2_ref, bh2_ref,
                o_ref):
    uL = uLt_ref[...]
    uR = uRt_ref[...]
    diff = uL - uR
    feats = (uL, uR, diff, jnp.abs(diff), (uL + uR) * 0.5, jnp.sign(diff))
    Win = Win_ref[...]
    h3 = feats[0][:, :, None] * Win[0][None, None, :]
    for f in range(1, 6):
        h3 = h3 + feats[f][:, :, None] * Win[f][None, None, :]
    kk, bb, hh = h3.shape
    h = h3.reshape(kk * bb, hh) + bin_ref[...]
    h = _gelu(_ln(h, gin_ref[...], bein_ref[...]))
    for (g_r, be_r, Wa_r, ba_r, Wb_r, bb_r) in (
            (g1_ref, be1_ref, W1a_ref, b1a_ref, W1b_ref, b1b_ref),
            (g2_ref, be2_ref, W2a_ref, b2a_ref, W2b_ref, b2b_ref),
            (g3_ref, be3_ref, W3a_ref, b3a_ref, W3b_ref, b3b_ref)):
        r = _ln(h, g_r[...], be_r[...])
        r = _gelu(jnp.dot(r, Wa_r[...], preferred_element_type=jnp.float32) + ba_r[...])
        r = jnp.dot(r, Wb_r[...], preferred_element_type=jnp.float32) + bb_r[...]
        h = h + r
    o = _ln(h, gh_ref[...], beh_ref[...])
    o = _gelu(jnp.dot(o, Wh1_ref[...], preferred_element_type=jnp.float32) + bh1_ref[...])
    o = jnp.dot(o, Wh2_ref[...], preferred_element_type=jnp.float32) + bh2_ref[...]
    o_ref[...] = o


def kernel(x, W_in, b_in, g_in, be_in, g1, be1, W1a, b1a, W1b, b1b,
           g2, be2, W2a, b2a, W2b, b2b, g3, be3, W3a, b3a, W3b, b3b,
           g_h, be_h, Wh1, bh1, Wh2, bh2):
    B, C, X = x.shape
    density = x[:, 0, :]
    coords = x[:, C - 1, :]
    f32 = jnp.float32

    front = pl.kernel(
        _front_sc,
        out_type=[jax.ShapeDtypeStruct((B, _L), f32)] * 4
        + [jax.ShapeDtypeStruct((B, _L), jnp.int32)],
        mesh=plsc.VectorSubcoreMesh(core_axis_name="c", subcore_axis_name="s",
                                    num_cores=2, num_subcores=16),
        compiler_params=pltpu.CompilerParams(needs_layout_passes=False),
        scratch_types=[pltpu.VMEM((_X + _L,), f32)] * 8
        + [pltpu.VMEM((_RPW, _L), f32)] * 4
        + [pltpu.VMEM((_RPW, _L), jnp.int32), pltpu.SemaphoreType.DMA],
    )
    uL16, uR16, fc16, vd16, cnt16 = front(density, coords)
    uL = uL16[:, :_K]
    uR = uR16[:, :_K]
    fc = fc16[:, :_K]
    valid = vd16[:, :_K]
    front_count = cnt16[:, 0]

    uLt = uL.T
    uRt = uR.T
    Wh1p = jnp.pad(Wh1, ((0, 0), (0, _H - Wh1.shape[1])))
    bh1p = jnp.pad(bh1, (0, _H - bh1.shape[0]))
    Wh2p = jnp.pad(Wh2, ((0, _H - Wh2.shape[0]), (0, _H - Wh2.shape[1])))
    bh2p = jnp.pad(bh2, (0, _H - bh2.shape[0]))
    row = lambda v: v.reshape(1, -1)
    o_pad = pl.pallas_call(
        _mlp_kernel,
        out_shape=jax.ShapeDtypeStruct((_K * B, _H), f32),
    )(uLt, uRt, W_in, row(b_in), row(g_in), row(be_in),
      row(g1), row(be1), W1a, row(b1a), W1b, row(b1b),
      row(g2), row(be2), W2a, row(b2a), W2b, row(b2b),
      row(g3), row(be3), W3a, row(b3a), W3b, row(b3b),
      row(g_h), row(be_h), Wh1p, row(bh1p), Wh2p, row(bh2p))

    fp_mlp = o_pad[:, :2].reshape(_K, B, 2).transpose(1, 2, 0)
    fp = jnp.concatenate(
        [fp_mlp, uL[:, None, :], uR[:, None, :], fc[:, None, :], valid[:, None, :]],
        axis=1)
    return (fp, front_count)


# speedup vs baseline: 2.9638x; 1.4554x over previous
import functools

import jax
import jax.numpy as jnp
from jax import lax
from jax.experimental import pallas as pl
from jax.experimental.pallas import tpu as pltpu
from jax.experimental.pallas import tpu_sc as plsc

_H = 128
_K = 8
_THR = 1e-06
_X = 8192
_L = 16
_NW = 32
_RPW = 4


def _front_sc(x_hbm, uL_hbm, uR_hbm, fc_hbm, vd_hbm, cnt_hbm,
              d0_r, d1_r, d2_r, d3_r, c0_r, c1_r, c2_r, c3_r,
              uL_st, uR_st, fc_st, vd_st, cnt_st, sem):
    wid = lax.axis_index("s") * 2 + lax.axis_index("c")
    base = wid * _RPW
    dbufs = (d0_r, d1_r, d2_r, d3_r)
    cbufs = (c0_r, c1_r, c2_r, c3_r)
    copies = []
    for r in range(_RPW):
        copies.append(pltpu.async_copy(x_hbm.at[base + r, 0], dbufs[r].at[pl.ds(0, _X)], sem))
        copies.append(pltpu.async_copy(x_hbm.at[base + r, 1], cbufs[r].at[pl.ds(0, _X)], sem))
    for cp in copies:
        cp.wait()

    inf = jnp.float32(jnp.inf)
    iota = lax.iota(jnp.int32, _L)
    lo8 = iota < _K

    def body(j, carry):
        bests, bidxs, cnts = carry
        nb, ni, nc = [], [], []
        off = j * _L
        for r in range(_RPW):
            d0 = dbufs[r][pl.ds(off, _L)]
            d1 = dbufs[r][pl.ds(off + 1, _L)]
            c0 = cbufs[r][pl.ds(off, _L)]
            c1 = cbufs[r][pl.ds(off + 1, _L)]
            gidx = off + iota
            disc = (jnp.abs(d0 - d1) > _THR) & (gidx < _X - 1)
            score = jnp.where(disc, (c0 + c1) * 0.5, inf)
            scand, sidx = plsc.sort_key_val(score, gidx)
            mk = jnp.where(lo8, bests[r], lax.rev(scand, (0,)))
            mi = jnp.where(lo8, bidxs[r], lax.rev(sidx, (0,)))
            b2, i2 = plsc.sort_key_val(mk, mi)
            nb.append(b2)
            ni.append(i2)
            nc.append(cnts[r] + disc.astype(jnp.int32))
        return (tuple(nb), tuple(ni), tuple(nc))

    init = (tuple(jnp.full((_L,), inf) for _ in range(_RPW)),
            tuple(jnp.zeros((_L,), jnp.int32) for _ in range(_RPW)),
            tuple(jnp.zeros((_L,), jnp.int32) for _ in range(_RPW)))
    bests, bidxs, cnts = lax.fori_loop(0, _X // _L, body, init)

    for r in range(_RPW):
        bv, bi, cv = bests[r], bidxs[r], cnts[r]
        uLg = plsc.load_gather(dbufs[r], [bi])
        uRg = plsc.load_gather(dbufs[r], [bi + 1])
        cLg = plsc.load_gather(cbufs[r], [bi])
        cRg = plsc.load_gather(cbufs[r], [bi + 1])
        uL_st[r, :] = uLg
        uR_st[r, :] = uRg
        fc_st[r, :] = (cLg + cRg) * 0.5
        vd_st[r, :] = jnp.where(bv < inf, jnp.ones((_L,), jnp.float32),
                                jnp.zeros((_L,), jnp.float32))
        cnt_st[r, :] = lax.broadcast(jnp.sum(cv), (_L,))

    pltpu.sync_copy(uL_st, uL_hbm.at[pl.ds(base, _RPW)])
    pltpu.sync_copy(uR_st, uR_hbm.at[pl.ds(base, _RPW)])
    pltpu.sync_copy(fc_st, fc_hbm.at[pl.ds(base, _RPW)])
    pltpu.sync_copy(vd_st, vd_hbm.at[pl.ds(base, _RPW)])
    pltpu.sync_copy(cnt_st, cnt_hbm.at[pl.ds(base, _RPW)])


def _ln(h, g, b):
    mu = jnp.mean(h, axis=-1, keepdims=True)
    var = jnp.mean((h - mu) ** 2, axis=-1, keepdims=True)
    return (h - mu) / jnp.sqrt(var + 1e-5) * g + b


def _gelu(h):
    return 0.5 * h * (1.0 + jax.lax.erf(h * 0.7071067811865476))


def _mlp_kernel(uLt_ref, uRt_ref, Win_ref, bin_ref, gin_ref, bein_ref,
                g1_ref, be1_ref, W1a_ref, b1a_ref, W1b_ref, b1b_ref,
                g2_ref, be2_ref, W2a_ref, b2a_ref, W2b_ref, b2b_ref,
                g3_ref, be3_ref, W3a_ref, b3a_ref, W3b_ref, b3b_ref,
                gh_ref, beh_ref, Wh1_ref, bh1_ref, Wh2_ref, bh2_ref,
                o_ref):
    uL = uLt_ref[...]
    uR = uRt_ref[...]
    diff = uL - uR
    feats = (uL, uR, diff, jnp.abs(diff), (uL + uR) * 0.5, jnp.sign(diff))
    Win = Win_ref[...]
    h3 = feats[0][:, :, None] * Win[0][None, None, :]
    for f in range(1, 6):
        h3 = h3 + feats[f][:, :, None] * Win[f][None, None, :]
    kk, bb, hh = h3.shape
    h = h3.reshape(kk * bb, hh) + bin_ref[...]
    h = _gelu(_ln(h, gin_ref[...], bein_ref[...]))
    for (g_r, be_r, Wa_r, ba_r, Wb_r, bb_r) in (
            (g1_ref, be1_ref, W1a_ref, b1a_ref, W1b_ref, b1b_ref),
            (g2_ref, be2_ref, W2a_ref, b2a_ref, W2b_ref, b2b_ref),
            (g3_ref, be3_ref, W3a_ref, b3a_ref, W3b_ref, b3b_ref)):
        r = _ln(h, g_r[...], be_r[...])
        r = _gelu(jnp.dot(r, Wa_r[...], preferred_element_type=jnp.float32) + ba_r[...])
        r = jnp.dot(r, Wb_r[...], preferred_element_type=jnp.float32) + bb_r[...]
        h = h + r
    o = _ln(h, gh_ref[...], beh_ref[...])
    o = _gelu(jnp.dot(o, Wh1_ref[...], preferred_element_type=jnp.float32) + bh1_ref[...])
    o = jnp.dot(o, Wh2_ref[...], preferred_element_type=jnp.float32) + bh2_ref[...]
    o_ref[...] = o


def kernel(x, W_in, b_in, g_in, be_in, g1, be1, W1a, b1a, W1b, b1b,
           g2, be2, W2a, b2a, W2b, b2b, g3, be3, W3a, b3a, W3b, b3b,
           g_h, be_h, Wh1, bh1, Wh2, bh2):
    B, C, X = x.shape
    f32 = jnp.float32

    front = pl.kernel(
        _front_sc,
        out_type=[jax.ShapeDtypeStruct((B, _L), f32)] * 4
        + [jax.ShapeDtypeStruct((B, _L), jnp.int32)],
        mesh=plsc.VectorSubcoreMesh(core_axis_name="c", subcore_axis_name="s",
                                    num_cores=2, num_subcores=16),
        compiler_params=pltpu.CompilerParams(needs_layout_passes=False),
        scratch_types=[pltpu.VMEM((_X + _L,), f32)] * 8
        + [pltpu.VMEM((_RPW, _L), f32)] * 4
        + [pltpu.VMEM((_RPW, _L), jnp.int32), pltpu.SemaphoreType.DMA],
    )
    uL16, uR16, fc16, vd16, cnt16 = front(x)
    uL = uL16[:, :_K]
    uR = uR16[:, :_K]
    fc = fc16[:, :_K]
    valid = vd16[:, :_K]
    front_count = cnt16[:, 0]

    uLt = uL.T
    uRt = uR.T
    Wh1p = jnp.pad(Wh1, ((0, 0), (0, _H - Wh1.shape[1])))
    bh1p = jnp.pad(bh1, (0, _H - bh1.shape[0]))
    Wh2p = jnp.pad(Wh2, ((0, _H - Wh2.shape[0]), (0, _H - Wh2.shape[1])))
    bh2p = jnp.pad(bh2, (0, _H - bh2.shape[0]))
    row = lambda v: v.reshape(1, -1)
    o_pad = pl.pallas_call(
        _mlp_kernel,
        out_shape=jax.ShapeDtypeStruct((_K * B, _H), f32),
    )(uLt, uRt, W_in, row(b_in), row(g_in), row(be_in),
      row(g1), row(be1), W1a, row(b1a), W1b, row(b1b),
      row(g2), row(be2), W2a, row(b2a), W2b, row(b2b),
      row(g3), row(be3), W3a, row(b3a), W3b, row(b3b),
      row(g_h), row(be_h), Wh1p, row(bh1p), Wh2p, row(bh2p))

    fp_mlp = o_pad[:, :2].reshape(_K, B, 2).transpose(1, 2, 0)
    fp = jnp.concatenate(
        [fp_mlp, uL[:, None, :], uR[:, None, :], fc[:, None, :], valid[:, None, :]],
        axis=1)
    return (fp, front_count)


# speedup vs baseline: 3.1573x; 1.0653x over previous
import functools

import jax
import jax.numpy as jnp
from jax import lax
from jax.experimental import pallas as pl
from jax.experimental.pallas import tpu as pltpu
from jax.experimental.pallas import tpu_sc as plsc

_H = 128
_K = 8
_THR = 1e-06
_X = 8192
_L = 16
_RPW = 4


def _front_sc(x_hbm, out_hbm,
              d0_r, d1_r, d2_r, d3_r, c0_r, c1_r, c2_r, c3_r,
              st, sem):
    wid = lax.axis_index("s") * 2 + lax.axis_index("c")
    base = wid * _RPW
    dbufs = (d0_r, d1_r, d2_r, d3_r)
    cbufs = (c0_r, c1_r, c2_r, c3_r)
    copies = []
    for r in range(_RPW):
        copies.append(pltpu.async_copy(x_hbm.at[base + r, 0], dbufs[r].at[pl.ds(0, _X)], sem))
        copies.append(pltpu.async_copy(x_hbm.at[base + r, 1], cbufs[r].at[pl.ds(0, _X)], sem))
    for cp in copies:
        cp.wait()

    inf = jnp.float32(jnp.inf)
    iota = lax.iota(jnp.int32, _L)
    lo8 = iota < _K

    def body(j, carry):
        bests, bidxs, cnts = carry
        nb, ni, nc = [], [], []
        off = j * _L
        for r in range(_RPW):
            d0 = dbufs[r][pl.ds(off, _L)]
            d1 = dbufs[r][pl.ds(off + 1, _L)]
            c0 = cbufs[r][pl.ds(off, _L)]
            c1 = cbufs[r][pl.ds(off + 1, _L)]
            gidx = off + iota
            disc = (jnp.abs(d0 - d1) > _THR) & (gidx < _X - 1)
            score = jnp.where(disc, (c0 + c1) * 0.5, inf)
            scand, sidx = plsc.sort_key_val(score, gidx)
            mk = jnp.where(lo8, bests[r], lax.rev(scand, (0,)))
            mi = jnp.where(lo8, bidxs[r], lax.rev(sidx, (0,)))
            b2, i2 = plsc.sort_key_val(mk, mi)
            nb.append(b2)
            ni.append(i2)
            nc.append(cnts[r] + disc.astype(jnp.int32))
        return (tuple(nb), tuple(ni), tuple(nc))

    init = (tuple(jnp.full((_L,), inf) for _ in range(_RPW)),
            tuple(jnp.zeros((_L,), jnp.int32) for _ in range(_RPW)),
            tuple(jnp.zeros((_L,), jnp.int32) for _ in range(_RPW)))
    bests, bidxs, cnts = lax.fori_loop(0, _X // _L, body, init)

    for r in range(_RPW):
        bv, bi, cv = bests[r], bidxs[r], cnts[r]
        uLg = plsc.load_gather(dbufs[r], [bi])
        uRg = plsc.load_gather(dbufs[r], [bi + 1])
        cLg = plsc.load_gather(cbufs[r], [bi])
        cRg = plsc.load_gather(cbufs[r], [bi + 1])
        st[r, pl.ds(0, _L)] = uLg
        st[r, pl.ds(_L, _L)] = uRg
        st[r, pl.ds(2 * _L, _L)] = (cLg + cRg) * 0.5
        st[r, pl.ds(3 * _L, _L)] = jnp.where(bv < inf, jnp.ones((_L,), jnp.float32),
                                             jnp.zeros((_L,), jnp.float32))
        st[r, pl.ds(4 * _L, _L)] = lax.broadcast(jnp.sum(cv).astype(jnp.float32), (_L,))

    pltpu.sync_copy(st, out_hbm.at[pl.ds(base, _RPW)])


def _ln(h, g, b):
    mu = jnp.mean(h, axis=-1, keepdims=True)
    var = jnp.mean((h - mu) ** 2, axis=-1, keepdims=True)
    return (h - mu) / jnp.sqrt(var + 1e-5) * g + b


def _gelu(h):
    return 0.5 * h * (1.0 + jax.lax.erf(h * 0.7071067811865476))


def _mlp_kernel(fr_ref, Win_ref, bin_ref, gin_ref, bein_ref,
                g1_ref, be1_ref, W1a_ref, b1a_ref, W1b_ref, b1b_ref,
                g2_ref, be2_ref, W2a_ref, b2a_ref, W2b_ref, b2b_ref,
                g3_ref, be3_ref, W3a_ref, b3a_ref, W3b_ref, b3b_ref,
                gh_ref, beh_ref, Wh1_ref, bh1_ref, Wh2_ref, bh2_ref,
                o_ref):
    uL = fr_ref[:, 0:_K]
    uR = fr_ref[:, _L:_L + _K]
    diff = uL - uR
    feats = (uL, uR, diff, jnp.abs(diff), (uL + uR) * 0.5, jnp.sign(diff))
    Win = Win_ref[...]
    h3 = feats[0][:, :, None] * Win[0][None, None, :]
    for f in range(1, 6):
        h3 = h3 + feats[f][:, :, None] * Win[f][None, None, :]
    bb, kk, hh = h3.shape
    h = h3.reshape(bb * kk, hh) + bin_ref[...]
    h = _gelu(_ln(h, gin_ref[...], bein_ref[...]))
    for (g_r, be_r, Wa_r, ba_r, Wb_r, bb_r) in (
            (g1_ref, be1_ref, W1a_ref, b1a_ref, W1b_ref, b1b_ref),
            (g2_ref, be2_ref, W2a_ref, b2a_ref, W2b_ref, b2b_ref),
            (g3_ref, be3_ref, W3a_ref, b3a_ref, W3b_ref, b3b_ref)):
        r = _ln(h, g_r[...], be_r[...])
        r = _gelu(jnp.dot(r, Wa_r[...], preferred_element_type=jnp.float32) + ba_r[...])
        r = jnp.dot(r, Wb_r[...], preferred_element_type=jnp.float32) + bb_r[...]
        h = h + r
    o = _ln(h, gh_ref[...], beh_ref[...])
    o = _gelu(jnp.dot(o, Wh1_ref[...], preferred_element_type=jnp.float32) + bh1_ref[...])
    o = jnp.dot(o, Wh2_ref[...], preferred_element_type=jnp.float32) + bh2_ref[...]
    o_ref[...] = o


def kernel(x, W_in, b_in, g_in, be_in, g1, be1, W1a, b1a, W1b, b1b,
           g2, be2, W2a, b2a, W2b, b2b, g3, be3, W3a, b3a, W3b, b3b,
           g_h, be_h, Wh1, bh1, Wh2, bh2):
    B, C, X = x.shape
    f32 = jnp.float32

    front = pl.kernel(
        _front_sc,
        out_type=jax.ShapeDtypeStruct((B, 5 * _L), f32),
        mesh=plsc.VectorSubcoreMesh(core_axis_name="c", subcore_axis_name="s",
                                    num_cores=2, num_subcores=16),
        compiler_params=pltpu.CompilerParams(needs_layout_passes=False),
        scratch_types=[pltpu.VMEM((_X + _L,), f32)] * 8
        + [pltpu.VMEM((_RPW, 5 * _L), f32), pltpu.SemaphoreType.DMA],
    )
    fr = front(x)

    Wh1p = jnp.pad(Wh1, ((0, 0), (0, _H - Wh1.shape[1])))
    bh1p = jnp.pad(bh1, (0, _H - bh1.shape[0]))
    Wh2p = jnp.pad(Wh2, ((0, _H - Wh2.shape[0]), (0, _H - Wh2.shape[1])))
    bh2p = jnp.pad(bh2, (0, _H - bh2.shape[0]))
    row = lambda v: v.reshape(1, -1)
    o_pad = pl.pallas_call(
        _mlp_kernel,
        out_shape=jax.ShapeDtypeStruct((B * _K, _H), f32),
    )(fr, W_in, row(b_in), row(g_in), row(be_in),
      row(g1), row(be1), W1a, row(b1a), W1b, row(b1b),
      row(g2), row(be2), W2a, row(b2a), W2b, row(b2b),
      row(g3), row(be3), W3a, row(b3a), W3b, row(b3b),
      row(g_h), row(be_h), Wh1p, row(bh1p), Wh2p, row(bh2p))

    fp_mlp = o_pad[:, :2].reshape(B, _K, 2).transpose(0, 2, 1)
    fr4 = jnp.stack([fr[:, p * _L:p * _L + _K] for p in range(4)], axis=1)
    fp = jnp.concatenate([fp_mlp, fr4], axis=1)
    front_count = fr[:, 4 * _L].astype(jnp.int32)
    return (fp, front_count)
